# baseline (speedup 1.0000x reference)
import functools

import jax
import jax.numpy as jnp
from jax import lax
from jax.experimental import pallas as pl
from jax.experimental.pallas import tpu as pltpu
from jax.experimental.pallas import tpu_sc as plsc

B = 16384
F = 26
V = 100000
D = 16
H = 128
BF = B * F
EMBED_OUT = F * D
TOTAL_ROWS = F * V

NC = 2
NS = 16
NW = NC * NS

RPW = B // NW
IPW = RPW * F
GCH = 128
NG = IPW // GCH
GROUPS = IPW // 16
OFF_PERIOD_GROUPS = 13
OFF_PERIOD = OFF_PERIOD_GROUPS * 16


CW = 1024
NBLK = TOTAL_ROWS // CW
TAIL = TOTAL_ROWS - NBLK * CW
TR_BASE = NBLK // NW
TR_EXTRA = NBLK - TR_BASE * NW


def _sc_transpose(embT_hbm, tail_hbm, out_hbm,
                  blk_v0, blk_v1, row_v0, row_v1,
                  sem_i0, sem_i1, sem_o0, sem_o1):
    w = lax.axis_index("s") * NC + lax.axis_index("c")
    nblk = TR_BASE + jnp.where(w < TR_EXTRA, 1, 0)
    start = w * TR_BASE + jnp.minimum(w, TR_EXTRA)
    iot16 = lax.iota(jnp.int32, 16) * D

    blks = (blk_v0, blk_v1)
    rows = (row_v0, row_v1)
    semi = (sem_i0, sem_i1)
    semo = (sem_o0, sem_o1)

    def start_in(b, slot):
        c0 = (start + b) * CW
        pltpu.async_copy(embT_hbm.at[:, pl.ds(c0, CW)], blks[slot], semi[slot])

    def step(b, slot):
        blk, row = blks[slot], rows[slot]
        pltpu.make_async_copy(embT_hbm.at[:, pl.ds(0, CW)], blk,
                              semi[slot]).wait()

        @pl.when(b >= 2)
        def _():
            pltpu.make_async_copy(row, out_hbm.at[pl.ds(0, CW * D)],
                                  semo[slot]).wait()

        def col_grp(c, _):
            cbase = c * 16
            for k in range(D):
                vals = blk[k, pl.ds(cbase, 16)]
                plsc.store_scatter(row, [iot16 + (cbase * D + k)], vals)
            return 0

        lax.fori_loop(0, CW // 16, col_grp, 0)

        @pl.when(b + 2 < nblk)
        def _():
            start_in(b + 2, slot)

        c0 = (start + b) * CW
        pltpu.async_copy(row, out_hbm.at[pl.ds(c0 * D, CW * D)], semo[slot])

    start_in(0, 0)
    start_in(1, 1)

    def pair(i, _):
        b = i * 2

        @pl.when(b < nblk)
        def _():
            step(b, 0)

        @pl.when(b + 1 < nblk)
        def _():
            step(b + 1, 1)

        return 0

    lax.fori_loop(0, (TR_BASE + 2) // 2, pair, 0,
                  unroll=False)

    pltpu.make_async_copy(rows[0], out_hbm.at[pl.ds(0, CW * D)], semo[0]).wait()
    pltpu.make_async_copy(rows[1], out_hbm.at[pl.ds(0, CW * D)], semo[1]).wait()

    @pl.when(w == NW - 1)
    def _():
        pltpu.sync_copy(tail_hbm, row_v0.at[pl.ds(0, TAIL * D)])
        pltpu.sync_copy(row_v0.at[pl.ds(0, TAIL * D)],
                        out_hbm.at[pl.ds(NBLK * CW * D, TAIL * D)])


def _tr_call(embT, tail_flat):
    mesh = plsc.VectorSubcoreMesh(core_axis_name="c", subcore_axis_name="s",
                                  num_cores=NC, num_subcores=NS)
    return pl.kernel(
        _sc_transpose,
        out_type=jax.ShapeDtypeStruct((TOTAL_ROWS * D,), jnp.float32),
        mesh=mesh,
        scratch_types=[
            pltpu.VMEM((D, CW), jnp.float32),
            pltpu.VMEM((D, CW), jnp.float32),
            pltpu.VMEM((CW * D,), jnp.float32),
            pltpu.VMEM((CW * D,), jnp.float32),
            pltpu.SemaphoreType.DMA,
            pltpu.SemaphoreType.DMA,
            pltpu.SemaphoreType.DMA,
            pltpu.SemaphoreType.DMA,
        ],
        compiler_params=pltpu.CompilerParams(use_tc_tiling_on_sc=True,
                                             needs_layout_passes=False),
    )(embT, tail_flat)


def _sc_gather(x_hbm, offs_hbm, emb_hbm, lin_hbm, out_hbm, linout_hbm,
               idx_v, offs_v, row_v, lin_v, sem_e, sem_l):
    wid = lax.axis_index("s") * NC + lax.axis_index("c")
    base_i = wid * IPW

    pltpu.sync_copy(x_hbm.at[pl.ds(base_i, IPW)], idx_v)
    pltpu.sync_copy(offs_hbm, offs_v)

    def add_body(i, _):
        for j in range(OFF_PERIOD_GROUPS):
            s = pl.ds(i * OFF_PERIOD + j * 16, 16)
            idx_v[s] = idx_v[s] + offs_v[pl.ds(j * 16, 16)]
        return 0

    lax.fori_loop(0, GROUPS // OFF_PERIOD_GROUPS, add_body, 0)

    def gather_body(g, _):
        isl = pl.ds(g * GCH, GCH)
        ce = pltpu.async_copy(emb_hbm.at[idx_v.at[isl]], row_v, sem_e)
        cl = pltpu.async_copy(lin_hbm.at[idx_v.at[isl]], lin_v.at[isl], sem_l)
        ce.wait()
        pltpu.sync_copy(row_v, out_hbm.at[pl.ds(base_i + g * GCH, GCH)])
        cl.wait()
        return 0

    lax.fori_loop(0, NG, gather_body, 0)

    pltpu.sync_copy(lin_v, linout_hbm.at[pl.ds(base_i, IPW)])


def _sc_call(x_flat, offs, emb, lin_flat):
    mesh = plsc.VectorSubcoreMesh(core_axis_name="c", subcore_axis_name="s",
                                  num_cores=NC, num_subcores=NS)
    return pl.kernel(
        _sc_gather,
        out_type=(jax.ShapeDtypeStruct((BF, D), jnp.float32),
                  jax.ShapeDtypeStruct((BF,), jnp.float32)),
        mesh=mesh,
        scratch_types=[
            pltpu.VMEM((IPW,), jnp.int32),
            pltpu.VMEM((OFF_PERIOD,), jnp.int32),
            pltpu.VMEM((GCH, D), jnp.float32),
            pltpu.VMEM((IPW,), jnp.float32),
            pltpu.SemaphoreType.DMA,
            pltpu.SemaphoreType.DMA,
        ],
        compiler_params=pltpu.CompilerParams(use_tc_tiling_on_sc=False),
    )(x_flat, offs, emb, lin_flat)


BB = 2048


def _mlp_body(flat_ref, lin_ref, w1_ref, b1_ref, w2t_ref, bias_ref, out_ref):
    h = jnp.dot(flat_ref[...], w1_ref[...], preferred_element_type=jnp.float32)
    h = jnp.maximum(h + b1_ref[...], 0.0)
    deep = jnp.sum(h * w2t_ref[...], axis=1, keepdims=True)
    wide = jnp.sum(lin_ref[...], axis=1, keepdims=True)
    out_ref[...] = jax.nn.sigmoid(deep + wide + bias_ref[...])


def _mlp_call(flat, linmat, W1, b1r, W2t, bias):
    grid = (B // BB,)
    return pl.pallas_call(
        _mlp_body,
        grid=grid,
        in_specs=[
            pl.BlockSpec((BB, EMBED_OUT), lambda i: (i, 0)),
            pl.BlockSpec((BB, F), lambda i: (i, 0)),
            pl.BlockSpec((EMBED_OUT, H), lambda i: (0, 0)),
            pl.BlockSpec((1, H), lambda i: (0, 0)),
            pl.BlockSpec((1, H), lambda i: (0, 0)),
            pl.BlockSpec((1, 1), lambda i: (0, 0)),
        ],
        out_specs=pl.BlockSpec((BB, 1), lambda i: (i, 0)),
        out_shape=jax.ShapeDtypeStruct((B, 1), jnp.float32),
    )(flat, linmat, W1, b1r, W2t, bias)


def kernel(x, emb, lin_w, lin_b, W1, b1, W2, b2):
    x_flat = x.astype(jnp.int32).reshape(BF)
    offs = ((jnp.arange(OFF_PERIOD, dtype=jnp.int32) % F) * V)
    lin_flat = lin_w.reshape(-1)
    tail_flat = emb[NBLK * CW:].reshape(TAIL * D)
    emb_rows = _tr_call(emb.T, tail_flat).reshape(TOTAL_ROWS, D)
    gathered, lin_gath = _sc_call(x_flat, offs, emb_rows, lin_flat)
    flat = gathered.reshape(B, EMBED_OUT)
    linmat = lin_gath.reshape(B, F)
    bias = (b2 + lin_b).reshape(1, 1)
    out = _mlp_call(flat, linmat, W1, b1.reshape(1, H), W2.reshape(1, H), bias)
    return out.reshape(B)


# speedup vs baseline: 1.4119x; 1.4119x over previous
import functools

import jax
import jax.numpy as jnp
from jax import lax
from jax.experimental import pallas as pl
from jax.experimental.pallas import tpu as pltpu
from jax.experimental.pallas import tpu_sc as plsc

B = 16384
F = 26
V = 100000
D = 16
H = 128
BF = B * F
EMBED_OUT = F * D
TOTAL_ROWS = F * V

NC = 2
NS = 16
NW = NC * NS

RPW = B // NW
IPW = RPW * F
GCH = 128
NG = IPW // GCH
GROUPS = IPW // 16
OFF_PERIOD_GROUPS = 13
OFF_PERIOD = OFF_PERIOD_GROUPS * 16


CW = 1024
NBLK = TOTAL_ROWS // CW
TAIL = TOTAL_ROWS - NBLK * CW
TR_BASE = NBLK // NW
TR_EXTRA = NBLK - TR_BASE * NW


def _sc_transpose(embT_hbm, tail_hbm, out_hbm,
                  blk_v0, blk_v1, row_v0, row_v1,
                  sem_i0, sem_i1, sem_o0, sem_o1):
    w = lax.axis_index("s") * NC + lax.axis_index("c")
    nblk = TR_BASE + jnp.where(w < TR_EXTRA, 1, 0)
    start = w * TR_BASE + jnp.minimum(w, TR_EXTRA)
    iot16 = lax.iota(jnp.int32, 16) * D

    blks = (blk_v0, blk_v1)
    rows = (row_v0, row_v1)
    semi = (sem_i0, sem_i1)
    semo = (sem_o0, sem_o1)

    def start_in(b, slot):
        c0 = (start + b) * CW
        pltpu.async_copy(embT_hbm.at[:, pl.ds(c0, CW)], blks[slot], semi[slot])

    def step(b, slot):
        blk, row = blks[slot], rows[slot]
        pltpu.make_async_copy(embT_hbm.at[:, pl.ds(0, CW)], blk,
                              semi[slot]).wait()

        @pl.when(b >= 2)
        def _():
            pltpu.make_async_copy(row, out_hbm.at[pl.ds(0, CW * D)],
                                  semo[slot]).wait()

        @plsc.parallel_loop(0, CW // 16, unroll=2)
        def col_grp(c):
            cbase = c * 16
            for k in range(D):
                vals = blk[k, pl.ds(cbase, 16)]
                plsc.store_scatter(row, [iot16 + (cbase * D + k)], vals)

        @pl.when(b + 2 < nblk)
        def _():
            start_in(b + 2, slot)

        c0 = (start + b) * CW
        pltpu.async_copy(row, out_hbm.at[pl.ds(c0 * D, CW * D)], semo[slot])

    start_in(0, 0)
    start_in(1, 1)

    def pair(i, _):
        b = i * 2

        @pl.when(b < nblk)
        def _():
            step(b, 0)

        @pl.when(b + 1 < nblk)
        def _():
            step(b + 1, 1)

        return 0

    lax.fori_loop(0, (TR_BASE + 2) // 2, pair, 0,
                  unroll=False)

    pltpu.make_async_copy(rows[0], out_hbm.at[pl.ds(0, CW * D)], semo[0]).wait()
    pltpu.make_async_copy(rows[1], out_hbm.at[pl.ds(0, CW * D)], semo[1]).wait()

    @pl.when(w == NW - 1)
    def _():
        pltpu.sync_copy(tail_hbm, row_v0.at[pl.ds(0, TAIL * D)])
        pltpu.sync_copy(row_v0.at[pl.ds(0, TAIL * D)],
                        out_hbm.at[pl.ds(NBLK * CW * D, TAIL * D)])


def _tr_call(embT, tail_flat):
    mesh = plsc.VectorSubcoreMesh(core_axis_name="c", subcore_axis_name="s",
                                  num_cores=NC, num_subcores=NS)
    return pl.kernel(
        _sc_transpose,
        out_type=jax.ShapeDtypeStruct((TOTAL_ROWS * D,), jnp.float32),
        mesh=mesh,
        scratch_types=[
            pltpu.VMEM((D, CW), jnp.float32),
            pltpu.VMEM((D, CW), jnp.float32),
            pltpu.VMEM((CW * D,), jnp.float32),
            pltpu.VMEM((CW * D,), jnp.float32),
            pltpu.SemaphoreType.DMA,
            pltpu.SemaphoreType.DMA,
            pltpu.SemaphoreType.DMA,
            pltpu.SemaphoreType.DMA,
        ],
        compiler_params=pltpu.CompilerParams(use_tc_tiling_on_sc=True,
                                             needs_layout_passes=False),
    )(embT, tail_flat)


def _sc_gather(x_hbm, offs_hbm, emb_hbm, lin_hbm, out_hbm, linout_hbm,
               idx_v, offs_v, row_v, lin_v, sem_e, sem_l):
    wid = lax.axis_index("s") * NC + lax.axis_index("c")
    base_i = wid * IPW

    pltpu.sync_copy(x_hbm.at[pl.ds(base_i, IPW)], idx_v)
    pltpu.sync_copy(offs_hbm, offs_v)

    def add_body(i, _):
        for j in range(OFF_PERIOD_GROUPS):
            s = pl.ds(i * OFF_PERIOD + j * 16, 16)
            idx_v[s] = idx_v[s] + offs_v[pl.ds(j * 16, 16)]
        return 0

    lax.fori_loop(0, GROUPS // OFF_PERIOD_GROUPS, add_body, 0)

    def gather_body(g, _):
        isl = pl.ds(g * GCH, GCH)
        ce = pltpu.async_copy(emb_hbm.at[idx_v.at[isl]], row_v, sem_e)
        cl = pltpu.async_copy(lin_hbm.at[idx_v.at[isl]], lin_v.at[isl], sem_l)
        ce.wait()
        pltpu.sync_copy(row_v, out_hbm.at[pl.ds(base_i + g * GCH, GCH)])
        cl.wait()
        return 0

    lax.fori_loop(0, NG, gather_body, 0)

    pltpu.sync_copy(lin_v, linout_hbm.at[pl.ds(base_i, IPW)])


def _sc_call(x_flat, offs, emb, lin_flat):
    mesh = plsc.VectorSubcoreMesh(core_axis_name="c", subcore_axis_name="s",
                                  num_cores=NC, num_subcores=NS)
    return pl.kernel(
        _sc_gather,
        out_type=(jax.ShapeDtypeStruct((BF, D), jnp.float32),
                  jax.ShapeDtypeStruct((BF,), jnp.float32)),
        mesh=mesh,
        scratch_types=[
            pltpu.VMEM((IPW,), jnp.int32),
            pltpu.VMEM((OFF_PERIOD,), jnp.int32),
            pltpu.VMEM((GCH, D), jnp.float32),
            pltpu.VMEM((IPW,), jnp.float32),
            pltpu.SemaphoreType.DMA,
            pltpu.SemaphoreType.DMA,
        ],
        compiler_params=pltpu.CompilerParams(use_tc_tiling_on_sc=False),
    )(x_flat, offs, emb, lin_flat)


BB = 2048


def _mlp_body(flat_ref, lin_ref, w1_ref, b1_ref, w2t_ref, bias_ref, out_ref):
    h = jnp.dot(flat_ref[...], w1_ref[...], preferred_element_type=jnp.float32)
    h = jnp.maximum(h + b1_ref[...], 0.0)
    deep = jnp.sum(h * w2t_ref[...], axis=1, keepdims=True)
    wide = jnp.sum(lin_ref[...], axis=1, keepdims=True)
    out_ref[...] = jax.nn.sigmoid(deep + wide + bias_ref[...])


def _mlp_call(flat, linmat, W1, b1r, W2t, bias):
    grid = (B // BB,)
    return pl.pallas_call(
        _mlp_body,
        grid=grid,
        in_specs=[
            pl.BlockSpec((BB, EMBED_OUT), lambda i: (i, 0)),
            pl.BlockSpec((BB, F), lambda i: (i, 0)),
            pl.BlockSpec((EMBED_OUT, H), lambda i: (0, 0)),
            pl.BlockSpec((1, H), lambda i: (0, 0)),
            pl.BlockSpec((1, H), lambda i: (0, 0)),
            pl.BlockSpec((1, 1), lambda i: (0, 0)),
        ],
        out_specs=pl.BlockSpec((BB, 1), lambda i: (i, 0)),
        out_shape=jax.ShapeDtypeStruct((B, 1), jnp.float32),
    )(flat, linmat, W1, b1r, W2t, bias)


def kernel(x, emb, lin_w, lin_b, W1, b1, W2, b2):
    x_flat = x.astype(jnp.int32).reshape(BF)
    offs = ((jnp.arange(OFF_PERIOD, dtype=jnp.int32) % F) * V)
    lin_flat = lin_w.reshape(-1)
    tail_flat = emb[NBLK * CW:].reshape(TAIL * D)
    emb_rows = _tr_call(emb.T, tail_flat).reshape(TOTAL_ROWS, D)
    gathered, lin_gath = _sc_call(x_flat, offs, emb_rows, lin_flat)
    flat = gathered.reshape(B, EMBED_OUT)
    linmat = lin_gath.reshape(B, F)
    bias = (b2 + lin_b).reshape(1, 1)
    out = _mlp_call(flat, linmat, W1, b1.reshape(1, H), W2.reshape(1, H), bias)
    return out.reshape(B)


# speedup vs baseline: 1.7021x; 1.2056x over previous
import functools

import jax
import jax.numpy as jnp
from jax import lax
from jax.experimental import pallas as pl
from jax.experimental.pallas import tpu as pltpu
from jax.experimental.pallas import tpu_sc as plsc

B = 16384
F = 26
V = 100000
D = 16
H = 128
BF = B * F
EMBED_OUT = F * D
TOTAL_ROWS = F * V

NC = 2
NS = 16
NW = NC * NS

RPW = B // NW
IPW = RPW * F
GCH = 128
NG = IPW // GCH
GROUPS = IPW // 16
OFF_PERIOD_GROUPS = 13
OFF_PERIOD = OFF_PERIOD_GROUPS * 16


CW = 1024
NBLK = TOTAL_ROWS // CW
TAIL = TOTAL_ROWS - NBLK * CW
TR_BASE = NBLK // NW
TR_EXTRA = NBLK - TR_BASE * NW


def _sc_transpose(embT_hbm, tail_hbm, out_hbm,
                  blk_v0, blk_v1, row_v0, row_v1,
                  sem_i0, sem_i1, sem_o0, sem_o1):
    w = lax.axis_index("s") * NC + lax.axis_index("c")
    nblk = TR_BASE + jnp.where(w < TR_EXTRA, 1, 0)
    start = w * TR_BASE + jnp.minimum(w, TR_EXTRA)
    iot16 = lax.iota(jnp.int32, 16) * D

    blks = (blk_v0, blk_v1)
    rows = (row_v0, row_v1)
    semi = (sem_i0, sem_i1)
    semo = (sem_o0, sem_o1)

    def start_in(b, slot):
        c0 = (start + b) * CW
        pltpu.async_copy(embT_hbm.at[:, pl.ds(c0, CW)], blks[slot], semi[slot])

    def step(b, slot):
        blk, row = blks[slot], rows[slot]
        pltpu.make_async_copy(embT_hbm.at[:, pl.ds(0, CW)], blk,
                              semi[slot]).wait()

        @pl.when(b >= 2)
        def _():
            pltpu.make_async_copy(row, out_hbm.at[pl.ds(0, CW * D)],
                                  semo[slot]).wait()

        @plsc.parallel_loop(0, CW // 16, unroll=2)
        def col_grp(c):
            cbase = c * 16
            for k in range(D):
                vals = blk[k, pl.ds(cbase, 16)]
                plsc.store_scatter(row, [iot16 + (cbase * D + k)], vals)

        @pl.when(b + 2 < nblk)
        def _():
            start_in(b + 2, slot)

        c0 = (start + b) * CW
        pltpu.async_copy(row, out_hbm.at[pl.ds(c0 * D, CW * D)], semo[slot])

    start_in(0, 0)
    start_in(1, 1)

    def pair(i, _):
        b = i * 2

        @pl.when(b < nblk)
        def _():
            step(b, 0)

        @pl.when(b + 1 < nblk)
        def _():
            step(b + 1, 1)

        return 0

    lax.fori_loop(0, (TR_BASE + 2) // 2, pair, 0,
                  unroll=False)

    pltpu.make_async_copy(rows[0], out_hbm.at[pl.ds(0, CW * D)], semo[0]).wait()
    pltpu.make_async_copy(rows[1], out_hbm.at[pl.ds(0, CW * D)], semo[1]).wait()

    @pl.when(w == NW - 1)
    def _():
        pltpu.sync_copy(tail_hbm, row_v0.at[pl.ds(0, TAIL * D)])
        pltpu.sync_copy(row_v0.at[pl.ds(0, TAIL * D)],
                        out_hbm.at[pl.ds(NBLK * CW * D, TAIL * D)])


def _tr_call(embT, tail_flat):
    mesh = plsc.VectorSubcoreMesh(core_axis_name="c", subcore_axis_name="s",
                                  num_cores=NC, num_subcores=NS)
    return pl.kernel(
        _sc_transpose,
        out_type=jax.ShapeDtypeStruct((TOTAL_ROWS * D,), jnp.float32),
        mesh=mesh,
        scratch_types=[
            pltpu.VMEM((D, CW), jnp.float32),
            pltpu.VMEM((D, CW), jnp.float32),
            pltpu.VMEM((CW * D,), jnp.float32),
            pltpu.VMEM((CW * D,), jnp.float32),
            pltpu.SemaphoreType.DMA,
            pltpu.SemaphoreType.DMA,
            pltpu.SemaphoreType.DMA,
            pltpu.SemaphoreType.DMA,
        ],
        compiler_params=pltpu.CompilerParams(use_tc_tiling_on_sc=True,
                                             needs_layout_passes=False),
    )(embT, tail_flat)


def _sc_gather(x_hbm, offs_hbm, emb_hbm, lin_hbm, out_hbm, linout_hbm,
               idx_v, offs_v, row_v0, row_v1, row_v2, row_v3, lin_v,
               sem_e0, sem_e1, sem_e2, sem_e3,
               sem_o0, sem_o1, sem_o2, sem_o3, sem_l):
    wid = lax.axis_index("s") * NC + lax.axis_index("c")
    base_i = wid * IPW
    rows = (row_v0, row_v1, row_v2, row_v3)
    seme = (sem_e0, sem_e1, sem_e2, sem_e3)
    semo = (sem_o0, sem_o1, sem_o2, sem_o3)

    pltpu.sync_copy(x_hbm.at[pl.ds(base_i, IPW)], idx_v)
    pltpu.sync_copy(offs_hbm, offs_v)

    @plsc.parallel_loop(0, GROUPS // OFF_PERIOD_GROUPS, unroll=2)
    def add_body(i):
        for j in range(OFF_PERIOD_GROUPS):
            s = pl.ds(i * OFF_PERIOD + j * 16, 16)
            idx_v[s] = idx_v[s] + offs_v[pl.ds(j * 16, 16)]

    def issue(g, j):
        pltpu.async_copy(emb_hbm.at[idx_v.at[pl.ds(g * GCH, GCH)]],
                         rows[j], seme[j])

    for j in range(4):
        issue(j, j)

    def gather_body(g, _):
        for j in range(4):
            gg = g * 4 + j
            pltpu.make_async_copy(emb_hbm.at[idx_v.at[pl.ds(0, GCH)]],
                                  rows[j], seme[j]).wait()
            pltpu.async_copy(lin_hbm.at[idx_v.at[pl.ds(gg * GCH, GCH)]],
                             lin_v.at[pl.ds(gg * GCH, GCH)], sem_l)
            pltpu.async_copy(rows[j], out_hbm.at[pl.ds(base_i + gg * GCH, GCH)],
                             semo[j])

            @pl.when(gg >= 8)
            def _():
                pltpu.make_async_copy(lin_hbm.at[pl.ds(0, GCH)],
                                      lin_v.at[pl.ds(0, GCH)], sem_l).wait()

            @pl.when(gg + 4 < NG)
            def _():
                pltpu.make_async_copy(rows[j], out_hbm.at[pl.ds(0, GCH)],
                                      semo[j]).wait()
                issue(gg + 4, j)

        return 0

    lax.fori_loop(0, NG // 4, gather_body, 0)

    for j in range(4):
        pltpu.make_async_copy(rows[j], out_hbm.at[pl.ds(0, GCH)],
                              semo[j]).wait()
    pltpu.make_async_copy(lin_hbm.at[pl.ds(0, 8 * GCH)],
                          lin_v.at[pl.ds(0, 8 * GCH)], sem_l).wait()

    pltpu.sync_copy(lin_v, linout_hbm.at[pl.ds(base_i, IPW)])


def _sc_call(x_flat, offs, emb, lin_flat):
    mesh = plsc.VectorSubcoreMesh(core_axis_name="c", subcore_axis_name="s",
                                  num_cores=NC, num_subcores=NS)
    return pl.kernel(
        _sc_gather,
        out_type=(jax.ShapeDtypeStruct((BF, D), jnp.float32),
                  jax.ShapeDtypeStruct((BF,), jnp.float32)),
        mesh=mesh,
        scratch_types=[
            pltpu.VMEM((IPW,), jnp.int32),
            pltpu.VMEM((OFF_PERIOD,), jnp.int32),
            pltpu.VMEM((GCH, D), jnp.float32),
            pltpu.VMEM((GCH, D), jnp.float32),
            pltpu.VMEM((GCH, D), jnp.float32),
            pltpu.VMEM((GCH, D), jnp.float32),
            pltpu.VMEM((IPW,), jnp.float32),
            pltpu.SemaphoreType.DMA,
            pltpu.SemaphoreType.DMA,
            pltpu.SemaphoreType.DMA,
            pltpu.SemaphoreType.DMA,
            pltpu.SemaphoreType.DMA,
            pltpu.SemaphoreType.DMA,
            pltpu.SemaphoreType.DMA,
            pltpu.SemaphoreType.DMA,
            pltpu.SemaphoreType.DMA,
        ],
        compiler_params=pltpu.CompilerParams(use_tc_tiling_on_sc=False),
    )(x_flat, offs, emb, lin_flat)


BB = 2048


def _mlp_body(flat_ref, lin_ref, w1_ref, b1_ref, w2t_ref, bias_ref, out_ref):
    h = jnp.dot(flat_ref[...], w1_ref[...], preferred_element_type=jnp.float32)
    h = jnp.maximum(h + b1_ref[...], 0.0)
    deep = jnp.sum(h * w2t_ref[...], axis=1, keepdims=True)
    wide = jnp.sum(lin_ref[...], axis=1, keepdims=True)
    out_ref[...] = jax.nn.sigmoid(deep + wide + bias_ref[...])


def _mlp_call(flat, linmat, W1, b1r, W2t, bias):
    grid = (B // BB,)
    return pl.pallas_call(
        _mlp_body,
        grid=grid,
        in_specs=[
            pl.BlockSpec((BB, EMBED_OUT), lambda i: (i, 0)),
            pl.BlockSpec((BB, F), lambda i: (i, 0)),
            pl.BlockSpec((EMBED_OUT, H), lambda i: (0, 0)),
            pl.BlockSpec((1, H), lambda i: (0, 0)),
            pl.BlockSpec((1, H), lambda i: (0, 0)),
            pl.BlockSpec((1, 1), lambda i: (0, 0)),
        ],
        out_specs=pl.BlockSpec((BB, 1), lambda i: (i, 0)),
        out_shape=jax.ShapeDtypeStruct((B, 1), jnp.float32),
    )(flat, linmat, W1, b1r, W2t, bias)


def kernel(x, emb, lin_w, lin_b, W1, b1, W2, b2):
    x_flat = x.astype(jnp.int32).reshape(BF)
    offs = ((jnp.arange(OFF_PERIOD, dtype=jnp.int32) % F) * V)
    lin_flat = lin_w.reshape(-1)
    tail_flat = emb[NBLK * CW:].reshape(TAIL * D)
    emb_rows = _tr_call(emb.T, tail_flat).reshape(TOTAL_ROWS, D)
    gathered, lin_gath = _sc_call(x_flat, offs, emb_rows, lin_flat)
    flat = gathered.reshape(B, EMBED_OUT)
    linmat = lin_gath.reshape(B, F)
    bias = (b2 + lin_b).reshape(1, 1)
    out = _mlp_call(flat, linmat, W1, b1.reshape(1, H), W2.reshape(1, H), bias)
    return out.reshape(B)


# speedup vs baseline: 1.7424x; 1.0237x over previous
import functools

import jax
import jax.numpy as jnp
from jax import lax
from jax.experimental import pallas as pl
from jax.experimental.pallas import tpu as pltpu
from jax.experimental.pallas import tpu_sc as plsc

B = 16384
F = 26
V = 100000
D = 16
H = 128
BF = B * F
EMBED_OUT = F * D
TOTAL_ROWS = F * V

NC = 2
NS = 16
NW = NC * NS

RPW = B // NW
IPW = RPW * F
GCH = 128
NG = IPW // GCH
GROUPS = IPW // 16
OFF_PERIOD_GROUPS = 13
OFF_PERIOD = OFF_PERIOD_GROUPS * 16


CW = 1024
NBLK = TOTAL_ROWS // CW
TAIL = TOTAL_ROWS - NBLK * CW
TR_BASE = NBLK // NW
TR_EXTRA = NBLK - TR_BASE * NW


def _sc_transpose(embT_hbm, tail_hbm, out_hbm,
                  blk_v0, blk_v1, row_v0, row_v1,
                  sem_i0, sem_i1, sem_o0, sem_o1):
    w = lax.axis_index("s") * NC + lax.axis_index("c")
    nblk = TR_BASE + jnp.where(w < TR_EXTRA, 1, 0)
    start = w * TR_BASE + jnp.minimum(w, TR_EXTRA)
    iot16 = lax.iota(jnp.int32, 16) * D

    blks = (blk_v0, blk_v1)
    rows = (row_v0, row_v1)
    semi = (sem_i0, sem_i1)
    semo = (sem_o0, sem_o1)

    def start_in(b, slot):
        c0 = (start + b) * CW
        pltpu.async_copy(embT_hbm.at[:, pl.ds(c0, CW)], blks[slot], semi[slot])

    def step(b, slot):
        blk, row = blks[slot], rows[slot]
        pltpu.make_async_copy(embT_hbm.at[:, pl.ds(0, CW)], blk,
                              semi[slot]).wait()

        @pl.when(b >= 2)
        def _():
            pltpu.make_async_copy(row, out_hbm.at[pl.ds(0, CW * D)],
                                  semo[slot]).wait()

        @plsc.parallel_loop(0, CW // 16, unroll=2)
        def col_grp(c):
            cbase = c * 16
            for k in range(D):
                vals = blk[k, pl.ds(cbase, 16)]
                plsc.store_scatter(row, [iot16 + (cbase * D + k)], vals)

        @pl.when(b + 2 < nblk)
        def _():
            start_in(b + 2, slot)

        c0 = (start + b) * CW
        pltpu.async_copy(row, out_hbm.at[pl.ds(c0 * D, CW * D)], semo[slot])

    start_in(0, 0)
    start_in(1, 1)

    def pair(i, _):
        b = i * 2

        @pl.when(b < nblk)
        def _():
            step(b, 0)

        @pl.when(b + 1 < nblk)
        def _():
            step(b + 1, 1)

        return 0

    lax.fori_loop(0, (TR_BASE + 2) // 2, pair, 0,
                  unroll=False)

    pltpu.make_async_copy(rows[0], out_hbm.at[pl.ds(0, CW * D)], semo[0]).wait()
    pltpu.make_async_copy(rows[1], out_hbm.at[pl.ds(0, CW * D)], semo[1]).wait()

    @pl.when(w == NW - 1)
    def _():
        pltpu.sync_copy(tail_hbm, row_v0.at[pl.ds(0, TAIL * D)])
        pltpu.sync_copy(row_v0.at[pl.ds(0, TAIL * D)],
                        out_hbm.at[pl.ds(NBLK * CW * D, TAIL * D)])


def _tr_call(embT, tail_flat):
    mesh = plsc.VectorSubcoreMesh(core_axis_name="c", subcore_axis_name="s",
                                  num_cores=NC, num_subcores=NS)
    return pl.kernel(
        _sc_transpose,
        out_type=jax.ShapeDtypeStruct((TOTAL_ROWS * D,), jnp.float32),
        mesh=mesh,
        scratch_types=[
            pltpu.VMEM((D, CW), jnp.float32),
            pltpu.VMEM((D, CW), jnp.float32),
            pltpu.VMEM((CW * D,), jnp.float32),
            pltpu.VMEM((CW * D,), jnp.float32),
            pltpu.SemaphoreType.DMA,
            pltpu.SemaphoreType.DMA,
            pltpu.SemaphoreType.DMA,
            pltpu.SemaphoreType.DMA,
        ],
        compiler_params=pltpu.CompilerParams(use_tc_tiling_on_sc=True,
                                             needs_layout_passes=False),
    )(embT, tail_flat)


def _sc_gather(x_hbm, offs_hbm, emb_hbm, lin_hbm, out_hbm, linout_hbm,
               idx_v, offs_v, row_v0, row_v1, row_v2, row_v3, lin_v, wide_v,
               sem_e0, sem_e1, sem_e2, sem_e3,
               sem_o0, sem_o1, sem_o2, sem_o3, sem_l):
    wid = lax.axis_index("s") * NC + lax.axis_index("c")
    base_i = wid * IPW
    rows = (row_v0, row_v1, row_v2, row_v3)
    seme = (sem_e0, sem_e1, sem_e2, sem_e3)
    semo = (sem_o0, sem_o1, sem_o2, sem_o3)

    pltpu.sync_copy(x_hbm.at[pl.ds(base_i, IPW)], idx_v)
    pltpu.sync_copy(offs_hbm, offs_v)

    @plsc.parallel_loop(0, GROUPS // OFF_PERIOD_GROUPS, unroll=2)
    def add_body(i):
        for j in range(OFF_PERIOD_GROUPS):
            s = pl.ds(i * OFF_PERIOD + j * 16, 16)
            idx_v[s] = idx_v[s] + offs_v[pl.ds(j * 16, 16)]

    def issue(g, j):
        pltpu.async_copy(emb_hbm.at[idx_v.at[pl.ds(g * GCH, GCH)]],
                         rows[j], seme[j])

    for j in range(4):
        issue(j, j)

    def gather_body(g, _):
        for j in range(4):
            gg = g * 4 + j
            pltpu.make_async_copy(emb_hbm.at[idx_v.at[pl.ds(0, GCH)]],
                                  rows[j], seme[j]).wait()
            pltpu.async_copy(lin_hbm.at[idx_v.at[pl.ds(gg * GCH, GCH)]],
                             lin_v.at[pl.ds(gg * GCH, GCH)], sem_l)
            pltpu.async_copy(rows[j], out_hbm.at[pl.ds(base_i + gg * GCH, GCH)],
                             semo[j])

            @pl.when(gg >= 8)
            def _():
                pltpu.make_async_copy(lin_hbm.at[pl.ds(0, GCH)],
                                      lin_v.at[pl.ds(0, GCH)], sem_l).wait()

            @pl.when(gg + 4 < NG)
            def _():
                pltpu.make_async_copy(rows[j], out_hbm.at[pl.ds(0, GCH)],
                                      semo[j]).wait()
                issue(gg + 4, j)

        return 0

    lax.fori_loop(0, NG // 4, gather_body, 0)

    for j in range(4):
        pltpu.make_async_copy(rows[j], out_hbm.at[pl.ds(0, GCH)],
                              semo[j]).wait()
    pltpu.make_async_copy(lin_hbm.at[pl.ds(0, 8 * GCH)],
                          lin_v.at[pl.ds(0, 8 * GCH)], sem_l).wait()

    row_idx = lax.iota(jnp.int32, 16) * F

    @plsc.parallel_loop(0, RPW // 16, unroll=2)
    def wide_body(grp):
        base = grp * (16 * F)
        acc = plsc.load_gather(lin_v, [row_idx + base])
        for f in range(1, F):
            acc = acc + plsc.load_gather(lin_v, [row_idx + (base + f)])
        wide_v[pl.ds(grp * 16, 16)] = acc

    pltpu.sync_copy(wide_v, linout_hbm.at[pl.ds(wid * RPW, RPW)])


def _sc_call(x_flat, offs, emb, lin_flat):
    mesh = plsc.VectorSubcoreMesh(core_axis_name="c", subcore_axis_name="s",
                                  num_cores=NC, num_subcores=NS)
    return pl.kernel(
        _sc_gather,
        out_type=(jax.ShapeDtypeStruct((BF, D), jnp.float32),
                  jax.ShapeDtypeStruct((B,), jnp.float32)),
        mesh=mesh,
        scratch_types=[
            pltpu.VMEM((IPW,), jnp.int32),
            pltpu.VMEM((OFF_PERIOD,), jnp.int32),
            pltpu.VMEM((GCH, D), jnp.float32),
            pltpu.VMEM((GCH, D), jnp.float32),
            pltpu.VMEM((GCH, D), jnp.float32),
            pltpu.VMEM((GCH, D), jnp.float32),
            pltpu.VMEM((IPW,), jnp.float32),
            pltpu.VMEM((RPW,), jnp.float32),
            pltpu.SemaphoreType.DMA,
            pltpu.SemaphoreType.DMA,
            pltpu.SemaphoreType.DMA,
            pltpu.SemaphoreType.DMA,
            pltpu.SemaphoreType.DMA,
            pltpu.SemaphoreType.DMA,
            pltpu.SemaphoreType.DMA,
            pltpu.SemaphoreType.DMA,
            pltpu.SemaphoreType.DMA,
        ],
        compiler_params=pltpu.CompilerParams(use_tc_tiling_on_sc=False,
                                             needs_layout_passes=False),
    )(x_flat, offs, emb, lin_flat)


BB = 2048


def _mlp_body(flat_ref, wide_ref, w1_ref, b1_ref, w2t_ref, bias_ref, out_ref):
    h = jnp.dot(flat_ref[...], w1_ref[...], preferred_element_type=jnp.float32)
    h = jnp.maximum(h + b1_ref[...], 0.0)
    deep = jnp.sum(h * w2t_ref[...], axis=1, keepdims=True)
    out_ref[...] = jax.nn.sigmoid(deep + wide_ref[...] + bias_ref[...])


def _mlp_call(flat, wide2d, W1, b1r, W2t, bias):
    grid = (B // BB,)
    return pl.pallas_call(
        _mlp_body,
        grid=grid,
        in_specs=[
            pl.BlockSpec((BB, EMBED_OUT), lambda i: (i, 0)),
            pl.BlockSpec((BB, 1), lambda i: (i, 0)),
            pl.BlockSpec((EMBED_OUT, H), lambda i: (0, 0)),
            pl.BlockSpec((1, H), lambda i: (0, 0)),
            pl.BlockSpec((1, H), lambda i: (0, 0)),
            pl.BlockSpec((1, 1), lambda i: (0, 0)),
        ],
        out_specs=pl.BlockSpec((BB, 1), lambda i: (i, 0)),
        out_shape=jax.ShapeDtypeStruct((B, 1), jnp.float32),
    )(flat, wide2d, W1, b1r, W2t, bias)


def kernel(x, emb, lin_w, lin_b, W1, b1, W2, b2):
    x_flat = x.astype(jnp.int32).reshape(BF)
    offs = ((jnp.arange(OFF_PERIOD, dtype=jnp.int32) % F) * V)
    lin_flat = lin_w.reshape(-1)
    tail_flat = emb[NBLK * CW:].reshape(TAIL * D)
    emb_rows = _tr_call(emb.T, tail_flat).reshape(TOTAL_ROWS, D)
    gathered, wide = _sc_call(x_flat, offs, emb_rows, lin_flat)
    flat = gathered.reshape(B, EMBED_OUT)
    wide2d = wide.reshape(B, 1)
    bias = (b2 + lin_b).reshape(1, 1)
    out = _mlp_call(flat, wide2d, W1, b1.reshape(1, H), W2.reshape(1, H), bias)
    return out.reshape(B)
